# baseline (speedup 1.0000x reference)
import functools

import jax
import jax.numpy as jnp
import numpy as np
from jax import lax
from jax.experimental import pallas as pl
from jax.experimental.pallas import tpu as pltpu
from jax.experimental.pallas import tpu_sc as plsc

N = 10000
E = 160000
DIN = 512
DOUT = 512

NCH = 4
CW = DOUT // NCH
NC = 2
NS = 16
L = 16

EPT = E // NS
K = 80
NB = EPT // K
ND = 4
NF = 2
NPAD = 10240
RPT = NPAD // NS
RD = 624
RT = N - RD * (NS - 1)
DRN = 32

RB = 1000


def _matmul_body(x_ref, w_ref, o_ref):
    o_ref[0] = jnp.dot(x_ref[...], w_ref[...],
                       preferred_element_type=jnp.float32
                       ).astype(jnp.bfloat16)


def _support_chunks(x, W):
    return pl.pallas_call(
        _matmul_body,
        grid=(N // RB, NCH),
        in_specs=[
            pl.BlockSpec((RB, DIN), lambda i, j: (i, 0)),
            pl.BlockSpec((DIN, CW), lambda i, j: (0, j)),
        ],
        out_specs=pl.BlockSpec((1, RB, CW), lambda i, j: (j, i, 0)),
        out_shape=jax.ShapeDtypeStruct((NCH, N, CW), jnp.bfloat16),
    )(x, W)


def _sc_spmm(sup4, src, dst, vals, b):
    mesh = plsc.VectorSubcoreMesh(core_axis_name="c", subcore_axis_name="s")

    @functools.partial(
        pl.kernel,
        out_type=jax.ShapeDtypeStruct((N, DOUT), jnp.float32),
        mesh=mesh,
        compiler_params=pltpu.CompilerParams(use_tc_tiling_on_sc=False,
                                             needs_layout_passes=False),
        scratch_types=(
            [pltpu.VMEM_SHARED((NPAD, CW), jnp.float32)]
            + [pltpu.VMEM((K,), jnp.int32) for _ in range(ND)]
            + [pltpu.VMEM((K // 2,), jnp.int32) for _ in range(2 * ND)]
            + [pltpu.VMEM((K + L,), jnp.float32) for _ in range(ND)]
            + [pltpu.VMEM((K, CW), jnp.bfloat16) for _ in range(ND)]
            + [pltpu.VMEM((K, CW), jnp.float32) for _ in range(NF)]
            + [pltpu.VMEM((DRN, CW), jnp.float32),
               pltpu.VMEM((CW,), jnp.float32)]
            + [pltpu.SemaphoreType.DMA for _ in range(3 * ND)]
        ),
    )
    def k(sup_ref, src_ref, dst_ref, val_ref, b_ref, out_ref, acc, *rest):
        srcb = rest[0:ND]
        dstbA = rest[ND:2 * ND]
        dstbB = rest[2 * ND:3 * ND]
        valb = rest[3 * ND:4 * ND]
        rowsbf = rest[4 * ND:5 * ND]
        rowsf = rest[5 * ND:5 * ND + NF]
        initb = rest[5 * ND + NF]
        biasb = rest[5 * ND + NF + 1]
        sbase = 5 * ND + NF + 2
        si = rest[sbase:sbase + ND]
        sg = rest[sbase + ND:sbase + 2 * ND]
        ss = rest[sbase + 2 * ND:sbase + 3 * ND]

        core = lax.axis_index("c")
        sid = lax.axis_index("s")

        for ch in range(NCH):
            @pl.when(core == ch // NC)
            def _chunk():
                ebase = sid * EPT

                pltpu.sync_copy(b_ref.at[pl.ds(ch * CW, CW)], biasb)

                bias_vecs = [biasb[pl.ds(j * L, L)] for j in range(CW // L)]

                @pl.loop(0, DRN)
                def _fill(r):
                    for j in range(CW // L):
                        initb[r, pl.ds(j * L, L)] = bias_vecs[j]

                @pl.loop(0, RPT // DRN)
                def _init(piece):
                    pltpu.sync_copy(
                        initb,
                        acc.at[pl.ds(sid * RPT + piece * DRN, DRN)])

                plsc.subcore_barrier()

                def idx_copies(bi, p):
                    return (
                        pltpu.make_async_copy(
                            src_ref.at[pl.ds(ebase + bi * K, K)],
                            srcb[p], si[p]),
                        pltpu.make_async_copy(
                            dst_ref.at[pl.ds(ebase + bi * K, K // 2)],
                            dstbA[p], si[p]),
                        pltpu.make_async_copy(
                            dst_ref.at[pl.ds(ebase + bi * K + K // 2,
                                             K // 2)],
                            dstbB[p], si[p]),
                        pltpu.make_async_copy(
                            val_ref.at[pl.ds(ebase + bi * K, K)],
                            valb[p].at[pl.ds(0, K)], si[p]),
                    )

                def issue_idx(bi, p):
                    for c in idx_copies(bi, p):
                        c.start()

                def wait_idx(bi, p):
                    for c in idx_copies(bi, p):
                        c.wait()

                H = K // 2

                def gather_copies(p):
                    return (
                        pltpu.make_async_copy(
                            sup_ref.at[ch].at[srcb[p].at[pl.ds(0, H)]],
                            rowsbf[p].at[pl.ds(0, H)], sg[p]),
                        pltpu.make_async_copy(
                            sup_ref.at[ch].at[srcb[p].at[pl.ds(H, H)]],
                            rowsbf[p].at[pl.ds(H, H)], sg[p]),
                    )

                def start_gather(bi, p):
                    for c in gather_copies(p):
                        c.start()

                def wait_gather(bi, p):
                    for c in gather_copies(p):
                        c.wait()

                def scatter_copies(p):
                    r32 = rowsf[p % NF]
                    return (
                        pltpu.make_async_copy(
                            r32.at[pl.ds(0, H)],
                            acc.at[dstbA[p]], ss[p]),
                        pltpu.make_async_copy(
                            r32.at[pl.ds(H, H)],
                            acc.at[dstbB[p]], ss[p]),
                    )

                def start_scatter(bi, p):
                    r32 = rowsf[p % NF]
                    pltpu.async_copy(r32.at[pl.ds(0, H)],
                                     acc.at[dstbA[p]], ss[p], add=True)
                    pltpu.async_copy(r32.at[pl.ds(H, H)],
                                     acc.at[dstbB[p]], ss[p], add=True)

                def wait_scatter(p):
                    for c in scatter_copies(p):
                        c.wait()

                def scale(bi, p):
                    rbf = rowsbf[p]
                    r32 = rowsf[p % NF]
                    vp = valb[p]

                    @pl.loop(0, K, unroll=4)
                    def _edge(e):
                        vvec = vp[pl.ds(e, L)]
                        vs = jnp.broadcast_to(vvec[0], (L,))
                        for c in range(CW // (2 * L)):
                            v2 = rbf[e, pl.ds(c * 2 * L, 2 * L)]
                            lo, hi = plsc.unpack(
                                v2, format=plsc.PackFormat.INTERLEAVED)
                            r32[e, pl.ds(c * 2 * L, L)] = lo * vs
                            r32[e, pl.ds(c * 2 * L + L, L)] = hi * vs

                def emit_batch(bi, p, in_loop):
                    p1 = (p + 1) % ND
                    p2 = (p + 2) % ND
                    if in_loop:
                        @pl.when(bi >= 2)
                        def _():
                            wait_scatter(p2)

                        @pl.when(bi + 2 < NB)
                        def _():
                            issue_idx(bi + 2, p2)

                        wait_idx(bi + 1, p1)
                        start_gather(bi + 1, p1)
                    else:
                        wait_scatter(p2)
                    wait_gather(bi, p)
                    scale(bi, p)
                    start_scatter(bi, p)

                issue_idx(0, 0)
                issue_idx(1, 1)
                wait_idx(0, 0)
                start_gather(0, 0)

                @pl.loop(0, (NB - 1) // ND)
                def _quad(g):
                    for b in range(ND):
                        emit_batch(ND * g + b, b, True)

                emit_batch(NB - 1, (NB - 1) % ND, False)
                wait_scatter((NB - 2) % ND)
                wait_scatter((NB - 1) % ND)

                plsc.subcore_barrier()

                @pl.when(sid < NS - 1)
                def _drain_main():
                    pltpu.sync_copy(
                        acc.at[pl.ds(sid * RD, RD)],
                        out_ref.at[pl.ds(sid * RD, RD),
                                   pl.ds(ch * CW, CW)])

                @pl.when(sid == NS - 1)
                def _drain_tail():
                    pltpu.sync_copy(
                        acc.at[pl.ds(RD * (NS - 1), RT)],
                        out_ref.at[pl.ds(RD * (NS - 1), RT),
                                   pl.ds(ch * CW, CW)])

    return k(sup4, src, dst, vals, b)


_PERM = np.empty((DOUT,), np.int32)
for _g in range(0, DOUT, 32):
    for _i in range(16):
        _PERM[_g + 2 * _i] = _g + _i
        _PERM[_g + 2 * _i + 1] = _g + 16 + _i


def kernel(input, adj_indices, adj_values, W, b):
    sup4 = _support_chunks(input, W[:, _PERM])
    dst = adj_indices[0]
    src = adj_indices[1]
    return _sc_spmm(sup4, src, dst, adj_values, b)


# speedup vs baseline: 1.7682x; 1.7682x over previous
import functools

import jax
import jax.numpy as jnp
from jax import lax
from jax.experimental import pallas as pl
from jax.experimental.pallas import tpu as pltpu
from jax.experimental.pallas import tpu_sc as plsc

N = 10000
E = 160000
DIN = 512
DOUT = 512

NCH = 4
CW = DOUT // NCH
NC = 2
NS = 16
L = 16

EPT = E // NS
K = 80
NB = EPT // K
ND = 4
NPAD = 10240
RPT = NPAD // NS
RD = 624
RT = N - RD * (NS - 1)
DRN = 32

RB = 1000


def _matmul_body(x_ref, w_ref, o_ref):
    o_ref[0] = jnp.dot(x_ref[...], w_ref[...],
                       preferred_element_type=jnp.float32)


def _support_chunks(x, W):
    return pl.pallas_call(
        _matmul_body,
        grid=(N // RB, NCH),
        in_specs=[
            pl.BlockSpec((RB, DIN), lambda i, j: (i, 0)),
            pl.BlockSpec((DIN, CW), lambda i, j: (0, j)),
        ],
        out_specs=pl.BlockSpec((1, RB, CW), lambda i, j: (j, i, 0)),
        out_shape=jax.ShapeDtypeStruct((NCH, N, CW), jnp.float32),
    )(x, W)


def _sc_spmm(sup4, src, dst, vals, b):
    mesh = plsc.VectorSubcoreMesh(core_axis_name="c", subcore_axis_name="s")

    @functools.partial(
        pl.kernel,
        out_type=jax.ShapeDtypeStruct((N, DOUT), jnp.float32),
        mesh=mesh,
        compiler_params=pltpu.CompilerParams(use_tc_tiling_on_sc=False),
        scratch_types=(
            [pltpu.VMEM_SHARED((NPAD, CW), jnp.float32)]
            + [pltpu.VMEM((K,), jnp.int32) for _ in range(ND)]
            + [pltpu.VMEM((K // 2,), jnp.int32) for _ in range(2 * ND)]
            + [pltpu.VMEM((K + L,), jnp.float32) for _ in range(ND)]
            + [pltpu.VMEM((K, CW), jnp.float32) for _ in range(ND)]
            + [pltpu.VMEM((DRN, CW), jnp.float32),
               pltpu.VMEM((CW,), jnp.float32)]
            + [pltpu.SemaphoreType.DMA for _ in range(3 * ND)]
        ),
    )
    def k(sup_ref, src_ref, dst_ref, val_ref, b_ref, out_ref, acc, *rest):
        srcb = rest[0:ND]
        dstbA = rest[ND:2 * ND]
        dstbB = rest[2 * ND:3 * ND]
        valb = rest[3 * ND:4 * ND]
        rows = rest[4 * ND:5 * ND]
        initb = rest[5 * ND]
        biasb = rest[5 * ND + 1]
        si = rest[5 * ND + 2:5 * ND + 2 + ND]
        sg = rest[5 * ND + 2 + ND:5 * ND + 2 + 2 * ND]
        ss = rest[5 * ND + 2 + 2 * ND:5 * ND + 2 + 3 * ND]

        core = lax.axis_index("c")
        sid = lax.axis_index("s")

        for ch in range(NCH):
            @pl.when(core == ch // NC)
            def _chunk():
                ebase = sid * EPT

                pltpu.sync_copy(b_ref.at[pl.ds(ch * CW, CW)], biasb)

                bias_vecs = [biasb[pl.ds(j * L, L)] for j in range(CW // L)]

                @pl.loop(0, DRN)
                def _fill(r):
                    for j in range(CW // L):
                        initb[r, pl.ds(j * L, L)] = bias_vecs[j]

                @pl.loop(0, RPT // DRN)
                def _init(piece):
                    pltpu.async_copy(
                        initb,
                        acc.at[pl.ds(sid * RPT + piece * DRN, DRN)],
                        si[0])

                @pl.loop(0, RPT // DRN)
                def _initwait(piece):
                    pltpu.make_async_copy(
                        initb,
                        acc.at[pl.ds(sid * RPT + piece * DRN, DRN)],
                        si[0]).wait()

                plsc.subcore_barrier()

                def idx_copies(bi, p):
                    return (
                        pltpu.make_async_copy(
                            src_ref.at[pl.ds(ebase + bi * K, K)],
                            srcb[p], si[p]),
                        pltpu.make_async_copy(
                            dst_ref.at[pl.ds(ebase + bi * K, K // 2)],
                            dstbA[p], si[p]),
                        pltpu.make_async_copy(
                            dst_ref.at[pl.ds(ebase + bi * K + K // 2,
                                             K // 2)],
                            dstbB[p], si[p]),
                        pltpu.make_async_copy(
                            val_ref.at[pl.ds(ebase + bi * K, K)],
                            valb[p].at[pl.ds(0, K)], si[p]),
                    )

                def issue_idx(bi, p):
                    for c in idx_copies(bi, p):
                        c.start()

                def wait_idx(bi, p):
                    for c in idx_copies(bi, p):
                        c.wait()

                H = K // 2

                def gather_copies(p):
                    return (
                        pltpu.make_async_copy(
                            sup_ref.at[ch].at[srcb[p].at[pl.ds(0, H)]],
                            rows[p].at[pl.ds(0, H)], sg[p]),
                        pltpu.make_async_copy(
                            sup_ref.at[ch].at[srcb[p].at[pl.ds(H, H)]],
                            rows[p].at[pl.ds(H, H)], sg[p]),
                    )

                def start_gather(bi, p):
                    for c in gather_copies(p):
                        c.start()

                def wait_gather(bi, p):
                    for c in gather_copies(p):
                        c.wait()

                def scatter_copies(p):
                    return (
                        pltpu.make_async_copy(
                            rows[p].at[pl.ds(0, H)],
                            acc.at[dstbA[p]], ss[p]),
                        pltpu.make_async_copy(
                            rows[p].at[pl.ds(H, H)],
                            acc.at[dstbB[p]], ss[p]),
                    )

                def start_scatter(bi, p):
                    pltpu.async_copy(rows[p].at[pl.ds(0, H)],
                                     acc.at[dstbA[p]], ss[p], add=True)
                    pltpu.async_copy(rows[p].at[pl.ds(H, H)],
                                     acc.at[dstbB[p]], ss[p], add=True)

                def wait_scatter(p):
                    for c in scatter_copies(p):
                        c.wait()

                def scale(bi, p):
                    rp = rows[p]
                    vp = valb[p]

                    @pl.loop(0, K, unroll=4)
                    def _edge(e):
                        vvec = vp[pl.ds(e, L)]
                        vs = jnp.broadcast_to(vvec[0], (L,))
                        for j in range(CW // L):
                            sl = pl.ds(j * L, L)
                            rp[e, sl] = rp[e, sl] * vs

                def emit_batch(bi, p, in_loop):
                    p1 = (p + 1) % ND
                    p2 = (p + 2) % ND
                    if in_loop:
                        @pl.when(bi + 2 < NB)
                        def _():
                            @pl.when(bi >= 2)
                            def _():
                                wait_scatter(p2)
                            issue_idx(bi + 2, p2)

                        wait_idx(bi + 1, p1)
                        start_gather(bi + 1, p1)
                    wait_gather(bi, p)
                    scale(bi, p)
                    start_scatter(bi, p)

                issue_idx(0, 0)
                issue_idx(1, 1)
                wait_idx(0, 0)
                start_gather(0, 0)

                @pl.loop(0, (NB - 1) // ND)
                def _quad(g):
                    for b in range(ND):
                        emit_batch(ND * g + b, b, True)

                emit_batch(NB - 1, (NB - 1) % ND, False)
                for p in range(ND):
                    wait_scatter(p)

                plsc.subcore_barrier()

                @pl.when(sid < NS - 1)
                def _drain_main():
                    pltpu.sync_copy(
                        acc.at[pl.ds(sid * RD, RD)],
                        out_ref.at[pl.ds(sid * RD, RD),
                                   pl.ds(ch * CW, CW)])

                @pl.when(sid == NS - 1)
                def _drain_tail():
                    pltpu.sync_copy(
                        acc.at[pl.ds(RD * (NS - 1), RT)],
                        out_ref.at[pl.ds(RD * (NS - 1), RT),
                                   pl.ds(ch * CW, CW)])

    return k(sup4, src, dst, vals, b)


def kernel(input, adj_indices, adj_values, W, b):
    sup4 = _support_chunks(input, W)
    dst = adj_indices[0]
    src = adj_indices[1]
    return _sc_spmm(sup4, src, dst, adj_values, b)
